# baseline (device time: 21949 ns/iter reference)
import jax
import jax.numpy as jnp
from jax import lax
from jax.experimental import pallas as pl
from jax.experimental.pallas import tpu as pltpu

N_CHUNKS = 16


def kernel(x):
    m_per, n = x.shape
    half = m_per // 2
    rc = half // N_CHUNKS

    def body(x_ref, out_ref, a_send, a_recv, b_send, b_recv, copy_sem):
        my_x = lax.axis_index("x")
        my_y = lax.axis_index("y")
        my_z = lax.axis_index("z")
        y_peer = (my_x, 1 - my_y, my_z)
        x_peer = (1 - my_x, my_y, my_z)

        barrier_sem = pltpu.get_barrier_semaphore()
        for nbr in (y_peer, x_peer):
            pl.semaphore_signal(
                barrier_sem, inc=1, device_id=nbr,
                device_id_type=pl.DeviceIdType.MESH,
            )
        pl.semaphore_wait(barrier_sem, 2)

        my_half = my_x * half
        own_base = my_y * m_per
        foreign_base = (1 - my_y) * m_per

        a_rdmas = []
        for c in range(N_CHUNKS):
            off = my_half + c * rc
            rdma = pltpu.make_async_remote_copy(
                src_ref=x_ref.at[pl.ds(off, rc), :],
                dst_ref=out_ref.at[pl.ds(own_base + off, rc), :],
                send_sem=a_send.at[c],
                recv_sem=a_recv.at[c],
                device_id=y_peer,
                device_id_type=pl.DeviceIdType.MESH,
            )
            rdma.start()
            a_rdmas.append(rdma)

        local_copy = pltpu.make_async_copy(
            x_ref, out_ref.at[pl.ds(own_base, m_per), :], copy_sem,
        )
        local_copy.start()

        b_rdmas = []
        for c in range(N_CHUNKS):
            roff = foreign_base + my_half + c * rc
            recv = pltpu.make_async_remote_copy(
                src_ref=out_ref.at[pl.ds(roff, rc), :],
                dst_ref=out_ref.at[pl.ds(roff, rc), :],
                send_sem=a_send.at[c],
                recv_sem=a_recv.at[c],
                device_id=y_peer,
                device_id_type=pl.DeviceIdType.MESH,
            )
            recv.wait_recv()
            fwd = pltpu.make_async_remote_copy(
                src_ref=out_ref.at[pl.ds(roff, rc), :],
                dst_ref=out_ref.at[pl.ds(roff, rc), :],
                send_sem=b_send.at[c],
                recv_sem=b_recv.at[c],
                device_id=x_peer,
                device_id_type=pl.DeviceIdType.MESH,
            )
            fwd.start()
            b_rdmas.append(fwd)

        other_half = (1 - my_x) * half
        for c in range(N_CHUNKS):
            roff = foreign_base + other_half + c * rc
            recv = pltpu.make_async_remote_copy(
                src_ref=out_ref.at[pl.ds(roff, rc), :],
                dst_ref=out_ref.at[pl.ds(roff, rc), :],
                send_sem=b_send.at[c],
                recv_sem=b_recv.at[c],
                device_id=x_peer,
                device_id_type=pl.DeviceIdType.MESH,
            )
            recv.wait_recv()
        local_copy.wait()
        for rdma in a_rdmas:
            rdma.wait_send()
        for rdma in b_rdmas:
            rdma.wait_send()

    return pl.pallas_call(
        body,
        out_shape=jax.ShapeDtypeStruct((2 * m_per, n), x.dtype),
        in_specs=[pl.BlockSpec(memory_space=pl.ANY)],
        out_specs=pl.BlockSpec(memory_space=pl.ANY),
        scratch_shapes=[
            pltpu.SemaphoreType.DMA((N_CHUNKS,)),
            pltpu.SemaphoreType.DMA((N_CHUNKS,)),
            pltpu.SemaphoreType.DMA((N_CHUNKS,)),
            pltpu.SemaphoreType.DMA((N_CHUNKS,)),
            pltpu.SemaphoreType.DMA,
        ],
        compiler_params=pltpu.CompilerParams(collective_id=0),
    )(x)


# device time: 20039 ns/iter; 1.0953x vs baseline; 1.0953x over previous
import jax
import jax.numpy as jnp
from jax import lax
from jax.experimental import pallas as pl
from jax.experimental.pallas import tpu as pltpu

RC = 32
N_HEAD = 13
TAIL = 96


def kernel(x):
    m_per, n = x.shape
    half = m_per // 2
    head = N_HEAD * RC
    assert head + TAIL == half

    def body(x_ref, out_ref, a_send, a_recv, b_send, b_recv, copy_sem):
        my_x = lax.axis_index("x")
        my_y = lax.axis_index("y")
        my_z = lax.axis_index("z")
        y_peer = (my_x, 1 - my_y, my_z)
        x_peer = (1 - my_x, my_y, my_z)

        barrier_sem = pltpu.get_barrier_semaphore()
        for nbr in (y_peer, x_peer):
            pl.semaphore_signal(
                barrier_sem, inc=1, device_id=nbr,
                device_id_type=pl.DeviceIdType.MESH,
            )
        pl.semaphore_wait(barrier_sem, 2)

        my_half = my_x * half
        other_half = (1 - my_x) * half
        own_base = my_y * m_per
        foreign_base = (1 - my_y) * m_per

        a_rdmas = []
        for c in range(N_HEAD):
            off = my_half + c * RC
            rdma = pltpu.make_async_remote_copy(
                src_ref=x_ref.at[pl.ds(off, RC), :],
                dst_ref=out_ref.at[pl.ds(own_base + off, RC), :],
                send_sem=a_send.at[c],
                recv_sem=a_recv.at[c],
                device_id=y_peer,
                device_id_type=pl.DeviceIdType.MESH,
            )
            rdma.start()
            a_rdmas.append(rdma)
        tail_off = other_half + head
        tail_rdma = pltpu.make_async_remote_copy(
            src_ref=x_ref.at[pl.ds(tail_off, TAIL), :],
            dst_ref=out_ref.at[pl.ds(own_base + tail_off, TAIL), :],
            send_sem=a_send.at[N_HEAD],
            recv_sem=a_recv.at[N_HEAD],
            device_id=y_peer,
            device_id_type=pl.DeviceIdType.MESH,
        )
        tail_rdma.start()
        a_rdmas.append(tail_rdma)

        local_copy = pltpu.make_async_copy(
            x_ref, out_ref.at[pl.ds(own_base, m_per), :], copy_sem,
        )
        local_copy.start()

        b_rdmas = []
        for c in range(N_HEAD):
            roff = foreign_base + my_half + c * RC
            recv = pltpu.make_async_remote_copy(
                src_ref=out_ref.at[pl.ds(roff, RC), :],
                dst_ref=out_ref.at[pl.ds(roff, RC), :],
                send_sem=a_send.at[c],
                recv_sem=a_recv.at[c],
                device_id=y_peer,
                device_id_type=pl.DeviceIdType.MESH,
            )
            recv.wait_recv()
            fwd = pltpu.make_async_remote_copy(
                src_ref=out_ref.at[pl.ds(roff, RC), :],
                dst_ref=out_ref.at[pl.ds(roff, RC), :],
                send_sem=b_send.at[c],
                recv_sem=b_recv.at[c],
                device_id=x_peer,
                device_id_type=pl.DeviceIdType.MESH,
            )
            fwd.start()
            b_rdmas.append(fwd)

        for c in range(N_HEAD):
            roff = foreign_base + other_half + c * RC
            recv = pltpu.make_async_remote_copy(
                src_ref=out_ref.at[pl.ds(roff, RC), :],
                dst_ref=out_ref.at[pl.ds(roff, RC), :],
                send_sem=b_send.at[c],
                recv_sem=b_recv.at[c],
                device_id=x_peer,
                device_id_type=pl.DeviceIdType.MESH,
            )
            recv.wait_recv()
        tail_recv = pltpu.make_async_remote_copy(
            src_ref=out_ref.at[pl.ds(foreign_base + tail_off, TAIL), :],
            dst_ref=out_ref.at[pl.ds(foreign_base + tail_off, TAIL), :],
            send_sem=a_send.at[N_HEAD],
            recv_sem=a_recv.at[N_HEAD],
            device_id=y_peer,
            device_id_type=pl.DeviceIdType.MESH,
        )
        tail_recv.wait_recv()
        local_copy.wait()
        for rdma in a_rdmas:
            rdma.wait_send()
        for rdma in b_rdmas:
            rdma.wait_send()

    return pl.pallas_call(
        body,
        out_shape=jax.ShapeDtypeStruct((2 * m_per, n), x.dtype),
        in_specs=[pl.BlockSpec(memory_space=pltpu.VMEM)],
        out_specs=pl.BlockSpec(memory_space=pltpu.VMEM),
        scratch_shapes=[
            pltpu.SemaphoreType.DMA((N_HEAD + 1,)),
            pltpu.SemaphoreType.DMA((N_HEAD + 1,)),
            pltpu.SemaphoreType.DMA((N_HEAD,)),
            pltpu.SemaphoreType.DMA((N_HEAD,)),
            pltpu.SemaphoreType.DMA,
        ],
        compiler_params=pltpu.CompilerParams(collective_id=0),
    )(x)
